# baseline (speedup 1.0000x reference)
import functools

import jax
import jax.numpy as jnp
from jax.experimental import pallas as pl
from jax.experimental.pallas import tpu as pltpu

_K = 1024
_D = 64
_BETA = 0.25
_NB = 8
_N = 1024
_NC = 1024
_LSCALE = (1.0 + _BETA) / (_NB * _N * _D)


def _vq_body(z_ref, e_ref, zq_ref, idx_ref, part_ref):
    zb = z_ref[0]
    e = e_ref[...]
    en = jnp.sum(e * e, axis=1, keepdims=True)
    zn = jnp.sum(zb * zb, axis=0, keepdims=True)
    ze = jnp.dot(e, zb, preferred_element_type=jnp.float32)
    dis = (zn - 2.0 * ze) + en
    minv = jnp.min(dis, axis=0, keepdims=True)
    iota = jax.lax.broadcasted_iota(jnp.int32, (_K, _NC), 0)
    idx = jnp.min(jnp.where(dis == minv, iota, _K), axis=0, keepdims=True)
    onehot = (iota == idx).astype(jnp.float32)
    zq = jax.lax.dot_general(e, onehot, (((0,), (0,)), ((), ())),
                             preferred_element_type=jnp.float32)
    zq_ref[0] = zb + (zq - zb)
    idx_ref[0] = idx
    part_ref[0, 0] = jnp.broadcast_to(jnp.sum(minv), (8, 128))


def kernel(z, embs):
    c = _N // _NC
    z3 = z.reshape(_NB, _D, _N)
    zq3, idx3, part = pl.pallas_call(
        _vq_body,
        grid=(_NB, c),
        in_specs=[
            pl.BlockSpec((1, _D, _NC), lambda b, j: (b, 0, j)),
            pl.BlockSpec((_K, _D), lambda b, j: (0, 0)),
        ],
        out_specs=[
            pl.BlockSpec((1, _D, _NC), lambda b, j: (b, 0, j)),
            pl.BlockSpec((1, 1, _NC), lambda b, j: (b, 0, j)),
            pl.BlockSpec((1, 1, 8, 128), lambda b, j: (b, j, 0, 0)),
        ],
        out_shape=[
            jax.ShapeDtypeStruct((_NB, _D, _N), jnp.float32),
            jax.ShapeDtypeStruct((_NB, 1, _N), jnp.int32),
            jax.ShapeDtypeStruct((_NB, c, 8, 128), jnp.float32),
        ],
        compiler_params=pltpu.CompilerParams(
            dimension_semantics=("parallel", "parallel")),
    )(z3, embs)
    z_q_out = zq3.reshape(_NB, _D, 32, 32)
    min_idxs = idx3.reshape(-1)
    loss = (1.0 + _BETA) * jnp.sum(part[:, :, 0, 0]) / (_NB * _N * _D)
    return (z_q_out, min_idxs, loss)


# speedup vs baseline: 1.1946x; 1.1946x over previous
import functools

import jax
import jax.numpy as jnp
from jax.experimental import pallas as pl
from jax.experimental.pallas import tpu as pltpu

_K = 1024
_D = 64
_BETA = 0.25
_NB = 8
_N = 1024
_NC = 1024
_LSCALE = (1.0 + _BETA) / (_NB * _N * _D)


def _vq_body(z_ref, e_ref, zq_ref, idx_ref, loss_ref, iota_ref):
    first = jnp.logical_and(pl.program_id(0) == 0, pl.program_id(1) == 0)

    @pl.when(first)
    def _():
        loss_ref[0, 0] = 0.0
        iota_ref[...] = jax.lax.broadcasted_iota(
            jnp.int32, (_K, _NC), 0).astype(jnp.float32)

    zb = z_ref[0]
    e = e_ref[...]
    en = jnp.sum(e * e, axis=1, keepdims=True)
    zn = jnp.sum(zb * zb, axis=0, keepdims=True)
    ze2 = jnp.dot(e + e, zb, preferred_element_type=jnp.float32)
    dis = (zn - ze2) + en
    minv = jnp.min(dis, axis=0, keepdims=True)
    iota = iota_ref[...]
    idx_f = jnp.min(jnp.where(dis == minv, iota, float(_K)),
                    axis=0, keepdims=True)
    onehot = (iota == idx_f).astype(jnp.float32)
    zq = jax.lax.dot_general(e, onehot, (((0,), (0,)), ((), ())),
                             preferred_element_type=jnp.float32)
    zq_ref[0] = zb + (zq - zb)
    idx_ref[0] = idx_f.astype(jnp.int32)
    loss_ref[0, 0] += _LSCALE * jnp.sum(minv)


def kernel(z, embs):
    c = _N // _NC
    z3 = z.reshape(_NB, _D, _N)
    zq3, idx3, loss = pl.pallas_call(
        _vq_body,
        grid=(_NB, c),
        in_specs=[
            pl.BlockSpec((1, _D, _NC), lambda b, j: (b, 0, j)),
            pl.BlockSpec((_K, _D), lambda b, j: (0, 0)),
        ],
        out_specs=[
            pl.BlockSpec((1, _D, _NC), lambda b, j: (b, 0, j)),
            pl.BlockSpec((1, 1, _NC), lambda b, j: (b, 0, j)),
            pl.BlockSpec((1, 1), lambda b, j: (0, 0),
                         memory_space=pltpu.SMEM),
        ],
        out_shape=[
            jax.ShapeDtypeStruct((_NB, _D, _N), jnp.float32),
            jax.ShapeDtypeStruct((_NB, 1, _N), jnp.int32),
            jax.ShapeDtypeStruct((1, 1), jnp.float32),
        ],
        scratch_shapes=[pltpu.VMEM((_K, _NC), jnp.float32)],
        compiler_params=pltpu.CompilerParams(
            dimension_semantics=("arbitrary", "arbitrary")),
    )(z3, embs)
    z_q_out = zq3.reshape(_NB, _D, 32, 32)
    min_idxs = idx3.reshape(-1)
    return (z_q_out, min_idxs, loss.reshape(()))
